# baseline (device time: 67719 ns/iter reference)
import jax
import jax.numpy as jnp
from jax import lax
from jax.experimental import pallas as pl
from jax.experimental.pallas import tpu as pltpu

N_DEV = 4
N_LAYERS = 3


def kernel(x, Win0, Wout0, Win1, Wout1, Win2, Wout2):
    m_per, d = x.shape
    M = N_DEV * m_per

    def body(x_ref, win0_ref, wout0_ref, win1_ref, wout1_ref, win2_ref,
             wout2_ref, out_ref, xg_ref, snd_ref, rcv_ref,
             ag_send, ag_recv, bf_send, bf_recv):
        my = lax.axis_index("i")
        left = (my - 1) % N_DEV
        right = (my + 1) % N_DEV

        barrier = pltpu.get_barrier_semaphore()
        for nbr in (left, right):
            pl.semaphore_signal(
                barrier, inc=1,
                device_id=(nbr,), device_id_type=pl.DeviceIdType.MESH,
            )
        pl.semaphore_wait(barrier, 2)

        xg_ref[pl.ds(my * m_per, m_per), :] = x_ref[:, :].astype(jnp.bfloat16)
        for h in range(N_DEV - 1):
            origin = (my - h) % N_DEV
            rdma = pltpu.make_async_remote_copy(
                src_ref=xg_ref.at[pl.ds(origin * m_per, m_per)],
                dst_ref=xg_ref.at[pl.ds(origin * m_per, m_per)],
                send_sem=ag_send.at[h],
                recv_sem=ag_recv.at[h],
                device_id=(right,),
                device_id_type=pl.DeviceIdType.MESH,
            )
            rdma.start()
            rdma.wait()

        weight_refs = (
            (win0_ref, wout0_ref),
            (win1_ref, wout1_ref),
            (win2_ref, wout2_ref),
        )
        xcur = xg_ref[:, :]
        for l, (win_ref, wout_ref) in enumerate(weight_refs):
            win = win_ref[:, :].astype(jnp.bfloat16)
            wout = wout_ref[:, :].astype(jnp.bfloat16)
            hid = jnp.dot(xcur, win, preferred_element_type=jnp.float32)
            hid = jnp.maximum(hid, 0.0).astype(jnp.bfloat16)
            acc = jnp.dot(hid, wout, preferred_element_type=jnp.float32)

            for step in range(2):
                e = 2 * l + step
                slot = e % 2
                if step == 0:
                    partner = jnp.where(my % 2 == 0, right, left)
                else:
                    partner = jnp.where(my % 2 == 0, left, right)
                snd_ref[:, :] = acc.astype(jnp.bfloat16)
                rdma = pltpu.make_async_remote_copy(
                    src_ref=snd_ref,
                    dst_ref=rcv_ref.at[slot],
                    send_sem=bf_send.at[e],
                    recv_sem=bf_recv.at[e],
                    device_id=(partner,),
                    device_id_type=pl.DeviceIdType.MESH,
                )
                rdma.start()
                rdma.wait()
                acc = acc + rcv_ref[slot, :, :].astype(jnp.float32)

            if l < N_LAYERS - 1:
                xcur = acc.astype(jnp.bfloat16)
            else:
                out_ref[:, :] = acc

    vmem = pl.BlockSpec(memory_space=pltpu.VMEM)
    return pl.pallas_call(
        body,
        out_shape=jax.ShapeDtypeStruct((M, d), jnp.float32),
        in_specs=[vmem] * 7,
        out_specs=vmem,
        scratch_shapes=[
            pltpu.VMEM((M, d), jnp.bfloat16),
            pltpu.VMEM((M, d), jnp.bfloat16),
            pltpu.VMEM((2, M, d), jnp.bfloat16),
            pltpu.SemaphoreType.DMA((N_DEV - 1,)),
            pltpu.SemaphoreType.DMA((N_DEV - 1,)),
            pltpu.SemaphoreType.DMA((2 * N_LAYERS,)),
            pltpu.SemaphoreType.DMA((2 * N_LAYERS,)),
        ],
        compiler_params=pltpu.CompilerParams(collective_id=0),
    )(x, Win0, Wout0, Win1, Wout1, Win2, Wout2)


# device time: 48236 ns/iter; 1.4039x vs baseline; 1.4039x over previous
import jax
import jax.numpy as jnp
from jax import lax
from jax.experimental import pallas as pl
from jax.experimental.pallas import tpu as pltpu

N_DEV = 4
N_LAYERS = 3
C = 4


def kernel(x, Win0, Wout0, Win1, Wout1, Win2, Wout2):
    m_per, d = x.shape
    M = N_DEV * m_per
    R = m_per

    def body(x_ref, win0_ref, wout0_ref, win1_ref, wout1_ref, win2_ref,
             wout2_ref, out_ref, xg_ref, sndA, rcvA, sndB, rcvB,
             ag_send, ag_recv, a_send, a_recv, b_send, b_recv):
        my = lax.axis_index("i")
        pA = my ^ 1
        pB = 3 - my

        barrier = pltpu.get_barrier_semaphore()
        for nbr in (pA, pB):
            pl.semaphore_signal(
                barrier, inc=1,
                device_id=(nbr,), device_id_type=pl.DeviceIdType.MESH,
            )
        pl.semaphore_wait(barrier, 2)

        def exch(snd, rcv, ssem, rsem, partner):
            return pltpu.make_async_remote_copy(
                src_ref=snd, dst_ref=rcv, send_sem=ssem, recv_sem=rsem,
                device_id=(partner,), device_id_type=pl.DeviceIdType.MESH,
            )

        xg_ref[pl.ds(my * R, R), :] = x_ref[:, :].astype(jnp.bfloat16)
        ag1 = exch(xg_ref.at[pl.ds(my * R, R)], xg_ref.at[pl.ds(my * R, R)],
                   ag_send.at[0], ag_recv.at[0], pA)
        ag1.start()
        ag1.wait()
        pair_lo = (my // 2) * 2 * R
        ag2 = exch(xg_ref.at[pl.ds(pair_lo, 2 * R)],
                   xg_ref.at[pl.ds(pair_lo, 2 * R)],
                   ag_send.at[1], ag_recv.at[1], pB)
        ag2.start()
        ag2.wait()

        weight_refs = (
            (win0_ref, wout0_ref),
            (win1_ref, wout1_ref),
            (win2_ref, wout2_ref),
        )
        partial = [None] * C
        pairsum_prev = [None] * C
        a_desc = [None] * C
        b_desc = [None] * C

        def finish_a_start_b(l, c):
            a_desc[c].wait()
            ps = partial[c] + rcvA[l, c, :, :].astype(jnp.float32)
            sndB[l, c, :, :] = ps.astype(jnp.bfloat16)
            bd = exch(sndB.at[l, c], rcvB.at[l, c],
                      b_send.at[l, c], b_recv.at[l, c], pB)
            bd.start()
            pairsum_prev[c] = ps
            b_desc[c] = bd

        for l, (win_ref, wout_ref) in enumerate(weight_refs):
            win = win_ref[:, :].astype(jnp.bfloat16)
            wout = wout_ref[:, :].astype(jnp.bfloat16)
            for c in range(C):
                if l == 0:
                    xin = xg_ref[pl.ds(c * R, R), :]
                else:
                    b_desc[c].wait()
                    xin = (pairsum_prev[c]
                           + rcvB[l - 1, c, :, :].astype(jnp.float32)
                           ).astype(jnp.bfloat16)
                hid = jnp.dot(xin, win, preferred_element_type=jnp.float32)
                hid = jnp.maximum(hid, 0.0).astype(jnp.bfloat16)
                p = jnp.dot(hid, wout, preferred_element_type=jnp.float32)
                partial[c] = p
                sndA[l, c, :, :] = p.astype(jnp.bfloat16)
                ad = exch(sndA.at[l, c], rcvA.at[l, c],
                          a_send.at[l, c], a_recv.at[l, c], pA)
                ad.start()
                a_desc[c] = ad
                if c >= 1:
                    finish_a_start_b(l, c - 1)
            finish_a_start_b(l, C - 1)

        for c in range(C):
            b_desc[c].wait()
            out_ref[pl.ds(c * R, R), :] = (
                pairsum_prev[c]
                + rcvB[N_LAYERS - 1, c, :, :].astype(jnp.float32)
            )

    vmem = pl.BlockSpec(memory_space=pltpu.VMEM)
    return pl.pallas_call(
        body,
        out_shape=jax.ShapeDtypeStruct((M, d), jnp.float32),
        in_specs=[vmem] * 7,
        out_specs=vmem,
        scratch_shapes=[
            pltpu.VMEM((M, d), jnp.bfloat16),
            pltpu.VMEM((N_LAYERS, C, R, d), jnp.bfloat16),
            pltpu.VMEM((N_LAYERS, C, R, d), jnp.bfloat16),
            pltpu.VMEM((N_LAYERS, C, R, d), jnp.bfloat16),
            pltpu.VMEM((N_LAYERS, C, R, d), jnp.bfloat16),
            pltpu.SemaphoreType.DMA((2,)),
            pltpu.SemaphoreType.DMA((2,)),
            pltpu.SemaphoreType.DMA((N_LAYERS, C)),
            pltpu.SemaphoreType.DMA((N_LAYERS, C)),
            pltpu.SemaphoreType.DMA((N_LAYERS, C)),
            pltpu.SemaphoreType.DMA((N_LAYERS, C)),
        ],
        compiler_params=pltpu.CompilerParams(collective_id=0),
    )(x, Win0, Wout0, Win1, Wout1, Win2, Wout2)


# device time: 43869 ns/iter; 1.5437x vs baseline; 1.0995x over previous
import jax
import jax.numpy as jnp
from jax import lax
from jax.experimental import pallas as pl
from jax.experimental.pallas import tpu as pltpu

N_DEV = 4
N_LAYERS = 3
C = 4
D = 2


def kernel(x, Win0, Wout0, Win1, Wout1, Win2, Wout2):
    m_per, d = x.shape
    M = N_DEV * m_per
    R = M // C

    def body(x_ref, win0_ref, wout0_ref, win1_ref, wout1_ref, win2_ref,
             wout2_ref, out_ref, xg_ref, sndA, rcvA, sndB, rcvB,
             ag_send, ag_recv, a_send, a_recv, b_send, b_recv):
        my = lax.axis_index("i")
        pA = my ^ 1
        pB = 3 - my
        pD = my ^ 2

        barrier = pltpu.get_barrier_semaphore()
        for nbr in (pA, pB, pD):
            pl.semaphore_signal(
                barrier, inc=1,
                device_id=(nbr,), device_id_type=pl.DeviceIdType.MESH,
            )
        pl.semaphore_wait(barrier, 3)

        def exch(snd, rcv, ssem, rsem, partner):
            return pltpu.make_async_remote_copy(
                src_ref=snd, dst_ref=rcv, send_sem=ssem, recv_sem=rsem,
                device_id=(partner,), device_id_type=pl.DeviceIdType.MESH,
            )

        xg_ref[pl.ds(my * m_per, m_per), :] = x_ref[:, :].astype(jnp.bfloat16)
        my_rows = xg_ref.at[pl.ds(my * m_per, m_per)]
        ag = []
        for k, peer in enumerate((pA, pB, pD)):
            g = exch(my_rows, my_rows, ag_send.at[k], ag_recv.at[k], peer)
            g.start()
            ag.append(g)
        for g in ag:
            g.wait_recv()

        weight_refs = (
            (win0_ref, wout0_ref),
            (win1_ref, wout1_ref),
            (win2_ref, wout2_ref),
        )
        partial = [None] * C
        pairsum_prev = [None] * C
        a_desc = [None] * C
        b_desc = [None] * C

        all_desc = []

        def finish_a_start_b(l, c):
            a_desc[c].wait_recv()
            ps = partial[c] + rcvA[l, c, :, :].astype(jnp.float32)
            sndB[l, c, :, :] = ps.astype(jnp.bfloat16)
            bd = exch(sndB.at[l, c], rcvB.at[l, c],
                      b_send.at[l, c], b_recv.at[l, c], pB)
            bd.start()
            pairsum_prev[c] = ps
            b_desc[c] = bd
            all_desc.append(bd)

        for l, (win_ref, wout_ref) in enumerate(weight_refs):
            win = win_ref[:, :].astype(jnp.bfloat16)
            wout = wout_ref[:, :].astype(jnp.bfloat16)
            for c in range(C):
                if l == 0:
                    xin = xg_ref[pl.ds(c * R, R), :]
                else:
                    b_desc[c].wait_recv()
                    xin = (pairsum_prev[c]
                           + rcvB[l - 1, c, :, :].astype(jnp.float32)
                           ).astype(jnp.bfloat16)
                hid = jnp.dot(xin, win, preferred_element_type=jnp.float32)
                hid = jnp.maximum(hid, 0.0).astype(jnp.bfloat16)
                p = jnp.dot(hid, wout, preferred_element_type=jnp.float32)
                partial[c] = p
                sndA[l, c, :, :] = p.astype(jnp.bfloat16)
                ad = exch(sndA.at[l, c], rcvA.at[l, c],
                          a_send.at[l, c], a_recv.at[l, c], pA)
                ad.start()
                a_desc[c] = ad
                all_desc.append(ad)
                if c >= D:
                    finish_a_start_b(l, c - D)
            for c in range(C - D, C):
                finish_a_start_b(l, c)

        for c in range(C):
            b_desc[c].wait_recv()
            out_ref[pl.ds(c * R, R), :] = (
                pairsum_prev[c]
                + rcvB[N_LAYERS - 1, c, :, :].astype(jnp.float32)
            ).astype(jnp.bfloat16)

        for g in ag:
            g.wait_send()
        for dsc in all_desc:
            dsc.wait_send()

    vmem = pl.BlockSpec(memory_space=pltpu.VMEM)
    return pl.pallas_call(
        body,
        out_shape=jax.ShapeDtypeStruct((M, d), jnp.bfloat16),
        in_specs=[vmem] * 7,
        out_specs=vmem,
        scratch_shapes=[
            pltpu.VMEM((M, d), jnp.bfloat16),
            pltpu.VMEM((N_LAYERS, C, R, d), jnp.bfloat16),
            pltpu.VMEM((N_LAYERS, C, R, d), jnp.bfloat16),
            pltpu.VMEM((N_LAYERS, C, R, d), jnp.bfloat16),
            pltpu.VMEM((N_LAYERS, C, R, d), jnp.bfloat16),
            pltpu.SemaphoreType.DMA((3,)),
            pltpu.SemaphoreType.DMA((3,)),
            pltpu.SemaphoreType.DMA((N_LAYERS, C)),
            pltpu.SemaphoreType.DMA((N_LAYERS, C)),
            pltpu.SemaphoreType.DMA((N_LAYERS, C)),
            pltpu.SemaphoreType.DMA((N_LAYERS, C)),
        ],
        compiler_params=pltpu.CompilerParams(collective_id=0),
    )(x, Win0, Wout0, Win1, Wout1, Win2, Wout2)
